# baseline (speedup 1.0000x reference)
import functools

import jax
import jax.numpy as jnp
from jax import lax
from jax.experimental import pallas as pl
from jax.experimental.pallas import tpu as pltpu
from jax.experimental.pallas import tpu_sc as plsc

N = 10000
E = 160000
D = 256
HD = 128
K = 64
NS = 16
G = 48
EPS = E // NS
VPS = EPS // 16
CAP = 10176
ZR = 40
PN = 10240
HALF = PN // 2
ACC = 5248
APS = ACC // NS
WPS = HALF // NS
DUMMY = 5130


def _perm(x, idx):
    return lax.gather(
        x, idx.reshape(16, 1),
        lax.GatherDimensionNumbers(offset_dims=(), collapsed_slice_dims=(0,),
                                   start_index_map=(0,)),
        (1,), mode=lax.GatherScatterMode.PROMISE_IN_BOUNDS)


def _lane0(x):
    return lax.squeeze(lax.slice(x, (0,), (1,)), (0,))


def _partition_body(src_hbm, dst_hbm, lists_hbm, counts_hbm,
                    srcv, dstv, lsv, cntv):
    c = lax.axis_index("c")
    s = lax.axis_index("s")

    pltpu.sync_copy(src_hbm.at[s], srcv)
    pltpu.sync_copy(dst_hbm.at[s], dstv)

    zeros16 = jnp.zeros((16,), jnp.int32)
    dummy16 = jnp.full((16,), DUMMY, jnp.int32)
    ones16 = jnp.full((16,), 1, jnp.int32)
    fifteen16 = jnp.full((16,), 15, jnp.int32)
    sixteen16 = jnp.full((16,), 16, jnp.int32)
    iota = lax.iota(jnp.int32, 16)

    def prefill(i, carry):
        off = pl.multiple_of(i * 16, 16)
        lsv[0, pl.ds(off, 16)] = zeros16
        lsv[1, pl.ds(off, 16)] = dummy16
        lsv[2, pl.ds(off, 16)] = zeros16
        lsv[3, pl.ds(off, 16)] = dummy16
        return carry

    lax.fori_loop(0, CAP // 16, prefill, 0)

    def binsearch(pref):
        idx = zeros16
        t = iota + 1
        for step in (8, 4, 2, 1):
            probe = jnp.minimum(idx + (step - 1), fifteen16)
            pv = _perm(pref, probe)
            idx = jnp.where(pv < t, idx + step, idx)
        return jnp.minimum(idx, fifteen16)

    def emit(buf_s, buf_d, n, r, pend_s, pend_d, g_s, g_d, pc):
        total = r + pc
        r_v = jnp.full((16,), r, jnp.int32)
        gg_idx = jnp.maximum(iota - r_v, zeros16)
        head = iota < r_v
        m_s = jnp.where(head, pend_s, _perm(g_s, gg_idx))
        m_d = jnp.where(head, pend_d, _perm(g_d, gg_idx))
        l_idx = jnp.minimum(iota + (sixteen16 - r_v), fifteen16)
        tot_v = jnp.full((16,), total, jnp.int32)
        l_s = _perm(g_s, l_idx)
        l_d = jnp.where(iota + 16 < tot_v, _perm(g_d, l_idx), dummy16)
        fi = total // 16

        @pl.when(fi > 0)
        def _():
            off = pl.ds(pl.multiple_of(n, 16), 16)
            buf_s[off] = m_s
            buf_d[off] = m_d

        fv = jnp.full((16,), fi, jnp.int32)
        return (n + fi * 16, total - fi * 16,
                m_s + (l_s - m_s) * fv, m_d + (l_d - m_d) * fv)

    lo_s, lo_d, hi_s, hi_d = lsv.at[0], lsv.at[1], lsv.at[2], lsv.at[3]

    def append(i, carry):
        nlo, rlo, pls, pld, nhi, rhi, phs, phd = carry
        off = pl.multiple_of(i * 16, 16)
        s16 = srcv[pl.ds(off, 16)]
        d16 = dstv[pl.ds(off, 16)]
        mlo = d16 < HALF
        inc = jnp.where(mlo, ones16, zeros16)
        p = inc
        for k in (1, 2, 4, 8):
            g = _perm(p, jnp.maximum(iota - k, zeros16))
            p = p + jnp.where(iota >= k, g, zeros16)
        pc_lo_v = _perm(p, fifteen16)
        pc_hi_v = sixteen16 - pc_lo_v
        pc_lo = lax.squeeze(lax.slice(p, (15,), (16,)), (0,))
        pc_hi = 16 - pc_lo
        p_hi = (iota + 1) - p
        ilo = binsearch(p)
        ihi = binsearch(p_hi)
        g_lo_s = _perm(s16, ilo)
        g_lo_d = jnp.where(iota < pc_lo_v, _perm(d16, ilo), dummy16)
        g_hi_s = _perm(s16, ihi)
        g_hi_d = jnp.where(iota < pc_hi_v, _perm(d16, ihi) - HALF, dummy16)
        nlo, rlo, pls, pld = emit(lo_s, lo_d, nlo, rlo, pls, pld,
                                  g_lo_s, g_lo_d, pc_lo)
        nhi, rhi, phs, phd = emit(hi_s, hi_d, nhi, rhi, phs, phd,
                                  g_hi_s, g_hi_d, pc_hi)
        return nlo, rlo, pls, pld, nhi, rhi, phs, phd

    z = jnp.int32(0)
    init = (z, z, zeros16, dummy16, z, z, zeros16, dummy16)
    nlo, rlo, pls, pld, nhi, rhi, phs, phd = lax.fori_loop(0, VPS, append, init)

    off = pl.ds(pl.multiple_of(nlo, 16), 16)
    lo_s[off] = pls
    lo_d[off] = pld
    off = pl.ds(pl.multiple_of(nhi, 16), 16)
    hi_s[off] = phs
    hi_d[off] = phd

    np_lo = (nlo + 16 + 4 * G - 1) // (4 * G)
    np_hi = (nhi + 16 + 4 * G - 1) // (4 * G)
    cntv[pl.ds(0, 16)] = jnp.full((16,), np_lo, jnp.int32)
    cntv[pl.ds(16, 16)] = jnp.full((16,), np_hi, jnp.int32)

    @pl.when(c == 0)
    def _():
        pltpu.sync_copy(lsv, lists_hbm.at[s])
        pltpu.sync_copy(cntv, counts_hbm.at[s])


_partition = pl.kernel(
    _partition_body,
    out_type=[jax.ShapeDtypeStruct((NS, 4, CAP), jnp.int32),
              jax.ShapeDtypeStruct((NS, 32), jnp.int32)],
    mesh=plsc.VectorSubcoreMesh(core_axis_name="c", subcore_axis_name="s"),
    scratch_types=[
        pltpu.VMEM((EPS,), jnp.int32),
        pltpu.VMEM((EPS,), jnp.int32),
        pltpu.VMEM((4, CAP), jnp.int32),
        pltpu.VMEM((32,), jnp.int32),
    ],
)


def _seg_sum_body(h_hbm, lists_hbm, counts_hbm, agg_hbm,
                  lsv, ldv, idx0, idx1, idx2, idx3, rows0, rows1, rows2,
                  rows3, zbuf, cntv, shared, sem0, sem1, sem2, sem3, zsem):
    c = lax.axis_index("c")
    s = lax.axis_index("s")

    zeros16 = jnp.zeros((16,), jnp.float32)

    def zrow(r, carry):
        for j in range(HD // 16):
            zbuf[r, pl.ds(j * 16, 16)] = zeros16
        return carry

    lax.fori_loop(0, ZR, zrow, 0)
    pltpu.sync_copy(counts_hbm.at[s], cntv)

    for p in range(2):
        zb = s * APS
        descs = []
        for j in range(APS // ZR):
            descs.append(pltpu.async_copy(
                zbuf, shared.at[pl.ds(zb + j * ZR, ZR)], zsem))
        descs.append(pltpu.async_copy(
            zbuf.at[pl.ds(0, APS % ZR)],
            shared.at[pl.ds(zb + (APS // ZR) * ZR, APS % ZR)], zsem))
        pltpu.sync_copy(lists_hbm.at[s].at[2 * p], lsv)
        pltpu.sync_copy(lists_hbm.at[s].at[2 * p + 1], ldv)
        for d in descs:
            d.wait()
        plsc.subcore_barrier()

        ntrip = _lane0(cntv[pl.ds(p * 16, 16)])
        nch = ntrip * 4

        def build_idx(idxv, kk):
            base = kk * G
            for j in range(G // 16):
                idxv[pl.ds(j * 16, 16)] = ldv[pl.ds(base + j * 16, 16)]

        def start_gather(kk, rowsv, sem):
            pltpu.async_copy(h_hbm.at[c].at[lsv.at[pl.ds(kk * G, G)]],
                             rowsv, sem)

        @pl.when(ntrip > 0)
        def _():
            start_gather(0, rows0, sem0)
            start_gather(1, rows1, sem1)
            start_gather(2, rows2, sem2)
            start_gather(3, rows3, sem3)

        def trip(i, carry):
            for b, (idxv, rowsv, sem) in enumerate(
                    ((idx0, rows0, sem0), (idx1, rows1, sem1),
                     (idx2, rows2, sem2), (idx3, rows3, sem3))):
                kk = 4 * i + b
                pltpu.make_async_copy(h_hbm.at[c].at[lsv.at[pl.ds(0, G)]],
                                      rowsv, sem).wait()
                build_idx(idxv, kk)
                pltpu.sync_copy(rowsv, shared.at[idxv], add=True)

                @pl.when(kk + 4 < nch)
                def _():
                    start_gather(kk + 4, rowsv, sem)
            return carry

        lax.fori_loop(0, ntrip, trip, 0)
        plsc.subcore_barrier()

        pltpu.sync_copy(shared.at[pl.ds(s * WPS, WPS)],
                        agg_hbm.at[c].at[pl.ds(p * HALF + s * WPS, WPS)])
        plsc.subcore_barrier()


_seg_sum = pl.kernel(
    _seg_sum_body,
    out_type=jax.ShapeDtypeStruct((2, PN, HD), jnp.float32),
    mesh=plsc.VectorSubcoreMesh(core_axis_name="c", subcore_axis_name="s"),
    scratch_types=[
        pltpu.VMEM((CAP,), jnp.int32),
        pltpu.VMEM((CAP,), jnp.int32),
        pltpu.VMEM((G,), jnp.int32),
        pltpu.VMEM((G,), jnp.int32),
        pltpu.VMEM((G,), jnp.int32),
        pltpu.VMEM((G,), jnp.int32),
        pltpu.VMEM((G, HD), jnp.float32),
        pltpu.VMEM((G, HD), jnp.float32),
        pltpu.VMEM((G, HD), jnp.float32),
        pltpu.VMEM((G, HD), jnp.float32),
        pltpu.VMEM((ZR, HD), jnp.float32),
        pltpu.VMEM((32,), jnp.int32),
        pltpu.VMEM_SHARED((ACC, HD), jnp.float32),
        pltpu.SemaphoreType.DMA,
        pltpu.SemaphoreType.DMA,
        pltpu.SemaphoreType.DMA,
        pltpu.SemaphoreType.DMA,
        pltpu.SemaphoreType.DMA,
    ],
)


BLK = 2000


def _gin_mlp_body(h, a, W1, b1, W2, b2, o, *, act_tanh):
    z = jnp.concatenate([h[0] + a[0], h[1] + a[1]], axis=1)
    t = jnp.dot(z, W1[...], preferred_element_type=jnp.float32) + b1[...]
    t = jnp.maximum(t, 0.0)
    t = jnp.dot(t, W2[...], preferred_element_type=jnp.float32) + b2[...]
    if act_tanh:
        t = jnp.tanh(t)
    o[0] = t[:, :HD]
    o[1] = t[:, HD:]


def _make_gin_mlp(act_tanh):
    pair = pl.BlockSpec((2, BLK, HD), lambda i: (0, i, 0))
    wspec = pl.BlockSpec((D, D), lambda i: (0, 0))
    bspec = pl.BlockSpec((1, D), lambda i: (0, 0))
    return pl.pallas_call(
        functools.partial(_gin_mlp_body, act_tanh=act_tanh),
        grid=(N // BLK,),
        in_specs=[pair, pair, wspec, bspec, wspec, bspec],
        out_specs=pair,
        out_shape=jax.ShapeDtypeStruct((2, N, HD), jnp.float32),
    )


_gin_mlp_tanh = _make_gin_mlp(True)


def _decoder_body(h, a, gW1, gb1, gW2, gb2,
                  dW1, db1, dW2, db2, dW3, db3, dW4, db4, dW5, db5,
                  gum, out):
    z = jnp.concatenate([h[0] + a[0], h[1] + a[1]], axis=1)
    t = jnp.maximum(jnp.dot(z, gW1[...], preferred_element_type=jnp.float32) + gb1[...], 0.0)
    t = jnp.dot(t, gW2[...], preferred_element_type=jnp.float32) + gb2[...]
    t = jnp.maximum(jnp.dot(t, dW1[...], preferred_element_type=jnp.float32) + db1[...], 0.0)
    t = jnp.maximum(jnp.dot(t, dW2[...], preferred_element_type=jnp.float32) + db2[...], 0.0)
    t = jnp.maximum(jnp.dot(t, dW3[...], preferred_element_type=jnp.float32) + db3[...], 0.0)
    t = jnp.dot(t, dW4[...], preferred_element_type=jnp.float32) + db4[...]
    t = jnp.dot(t, dW5[...], preferred_element_type=jnp.float32) + db5[...]
    t = t + gum[...]
    m = jnp.max(t, axis=1, keepdims=True)
    e = jnp.exp(t - m)
    out[...] = e / jnp.sum(e, axis=1, keepdims=True)


def _make_decoder():
    pair = pl.BlockSpec((2, BLK, HD), lambda i: (0, i, 0))
    w = pl.BlockSpec((D, D), lambda i: (0, 0))
    b = pl.BlockSpec((1, D), lambda i: (0, 0))
    wk = pl.BlockSpec((D, K), lambda i: (0, 0))
    bk = pl.BlockSpec((1, K), lambda i: (0, 0))
    wkk = pl.BlockSpec((K, K), lambda i: (0, 0))
    gspec = pl.BlockSpec((BLK, K), lambda i: (i, 0))
    return pl.pallas_call(
        _decoder_body,
        grid=(N // BLK,),
        in_specs=[pair, pair, w, b, w, b,
                  w, b, w, b, w, b, wk, bk, wkk, bk, gspec],
        out_specs=gspec,
        out_shape=jax.ShapeDtypeStruct((N, K), jnp.float32),
    )


_decoder = _make_decoder()


def kernel(x, edge_index, g1W1, g1b1, g1W2, g1b2, g2W1, g2b1, g2W2, g2b2,
           dW1, db1, dW2, db2, dW3, db3, dW4, db4, dW5, db5):
    src = edge_index[0].reshape(NS, EPS)
    dst = edge_index[1].reshape(NS, EPS)
    h = jnp.stack([x[:, :HD], x[:, HD:]], axis=0)

    lists, counts = _partition(src, dst)

    b1 = g1b1.reshape(1, D)
    b2 = g1b2.reshape(1, D)
    c1 = g2b1.reshape(1, D)
    c2 = g2b2.reshape(1, D)

    a = _seg_sum(h, lists, counts)
    h = _gin_mlp_tanh(h, a[:, :N], g1W1, b1, g1W2, b2)
    for _ in range(3):
        a = _seg_sum(h, lists, counts)
        h = _gin_mlp_tanh(h, a[:, :N], g2W1, c1, g2W2, c2)

    a = _seg_sum(h, lists, counts)
    u = jax.random.uniform(jax.random.key(42), (N, K), minval=1e-6, maxval=1.0 - 1e-6)
    gum = -jnp.log(-jnp.log(u))
    return _decoder(h, a[:, :N], g2W1, c1, g2W2, c2,
                    dW1, db1.reshape(1, D), dW2, db2.reshape(1, D),
                    dW3, db3.reshape(1, D), dW4, db4.reshape(1, K),
                    dW5, db5.reshape(1, K), gum)


# speedup vs baseline: 1.1095x; 1.1095x over previous
import functools

import jax
import jax.numpy as jnp
from jax import lax
from jax.experimental import pallas as pl
from jax.experimental.pallas import tpu as pltpu
from jax.experimental.pallas import tpu_sc as plsc

N = 10000
E = 160000
D = 256
HD = 128
K = 64
NS = 16
G = 48
EPS = E // NS
VPS = EPS // 16
CAP = 10080
ZR = 40
PN = 10240
HALF = PN // 2
ACC = 5248
APS = ACC // NS
WPS = HALF // NS
DUMMY = 5130


def _perm(x, idx):
    return lax.gather(
        x, idx.reshape(16, 1),
        lax.GatherDimensionNumbers(offset_dims=(), collapsed_slice_dims=(0,),
                                   start_index_map=(0,)),
        (1,), mode=lax.GatherScatterMode.PROMISE_IN_BOUNDS)


def _lane0(x):
    return lax.squeeze(lax.slice(x, (0,), (1,)), (0,))


def _partition_body(src_hbm, dst_hbm, lists_hbm, counts_hbm,
                    srcv, dstv, lsv, cntv):
    c = lax.axis_index("c")
    s = lax.axis_index("s")

    pltpu.sync_copy(src_hbm.at[s], srcv)
    pltpu.sync_copy(dst_hbm.at[s], dstv)

    zeros16 = jnp.zeros((16,), jnp.int32)
    dummy16 = jnp.full((16,), DUMMY, jnp.int32)
    ones16 = jnp.full((16,), 1, jnp.int32)
    fifteen16 = jnp.full((16,), 15, jnp.int32)
    sixteen16 = jnp.full((16,), 16, jnp.int32)
    iota = lax.iota(jnp.int32, 16)

    def prefill(i, carry):
        off = pl.multiple_of(i * 16, 16)
        lsv[0, pl.ds(off, 16)] = zeros16
        lsv[1, pl.ds(off, 16)] = dummy16
        lsv[2, pl.ds(off, 16)] = zeros16
        lsv[3, pl.ds(off, 16)] = dummy16
        return carry

    lax.fori_loop(0, CAP // 16, prefill, 0)

    def binsearch(pref):
        idx = zeros16
        t = iota + 1
        for step in (8, 4, 2, 1):
            probe = jnp.minimum(idx + (step - 1), fifteen16)
            pv = _perm(pref, probe)
            idx = jnp.where(pv < t, idx + step, idx)
        return jnp.minimum(idx, fifteen16)

    def emit(buf_s, buf_d, n, r, pend_s, pend_d, g_s, g_d, pc):
        total = r + pc
        r_v = jnp.full((16,), r, jnp.int32)
        gg_idx = jnp.maximum(iota - r_v, zeros16)
        head = iota < r_v
        m_s = jnp.where(head, pend_s, _perm(g_s, gg_idx))
        m_d = jnp.where(head, pend_d, _perm(g_d, gg_idx))
        l_idx = jnp.minimum(iota + (sixteen16 - r_v), fifteen16)
        tot_v = jnp.full((16,), total, jnp.int32)
        l_s = _perm(g_s, l_idx)
        l_d = jnp.where(iota + 16 < tot_v, _perm(g_d, l_idx), dummy16)
        fi = total // 16

        @pl.when(fi > 0)
        def _():
            off = pl.ds(pl.multiple_of(n, 16), 16)
            buf_s[off] = m_s
            buf_d[off] = m_d

        fv = jnp.full((16,), fi, jnp.int32)
        return (n + fi * 16, total - fi * 16,
                m_s + (l_s - m_s) * fv, m_d + (l_d - m_d) * fv)

    lo_s, lo_d, hi_s, hi_d = lsv.at[0], lsv.at[1], lsv.at[2], lsv.at[3]

    def append(i, carry):
        nlo, rlo, pls, pld, nhi, rhi, phs, phd = carry
        off = pl.multiple_of(i * 16, 16)
        s16 = srcv[pl.ds(off, 16)]
        d16 = dstv[pl.ds(off, 16)]
        mlo = d16 < HALF
        inc = jnp.where(mlo, ones16, zeros16)
        p = inc
        for k in (1, 2, 4, 8):
            g = _perm(p, jnp.maximum(iota - k, zeros16))
            p = p + jnp.where(iota >= k, g, zeros16)
        pc_lo_v = _perm(p, fifteen16)
        pc_hi_v = sixteen16 - pc_lo_v
        pc_lo = lax.squeeze(lax.slice(p, (15,), (16,)), (0,))
        pc_hi = 16 - pc_lo
        p_hi = (iota + 1) - p
        ilo = binsearch(p)
        ihi = binsearch(p_hi)
        g_lo_s = _perm(s16, ilo)
        g_lo_d = jnp.where(iota < pc_lo_v, _perm(d16, ilo), dummy16)
        g_hi_s = _perm(s16, ihi)
        g_hi_d = jnp.where(iota < pc_hi_v, _perm(d16, ihi) - HALF, dummy16)
        nlo, rlo, pls, pld = emit(lo_s, lo_d, nlo, rlo, pls, pld,
                                  g_lo_s, g_lo_d, pc_lo)
        nhi, rhi, phs, phd = emit(hi_s, hi_d, nhi, rhi, phs, phd,
                                  g_hi_s, g_hi_d, pc_hi)
        return nlo, rlo, pls, pld, nhi, rhi, phs, phd

    z = jnp.int32(0)
    init = (z, z, zeros16, dummy16, z, z, zeros16, dummy16)
    nlo, rlo, pls, pld, nhi, rhi, phs, phd = lax.fori_loop(0, VPS, append, init)

    off = pl.ds(pl.multiple_of(nlo, 16), 16)
    lo_s[off] = pls
    lo_d[off] = pld
    off = pl.ds(pl.multiple_of(nhi, 16), 16)
    hi_s[off] = phs
    hi_d[off] = phd

    np_lo = (nlo + 16 + 3 * G - 1) // (3 * G)
    np_hi = (nhi + 16 + 3 * G - 1) // (3 * G)
    cntv[pl.ds(0, 16)] = jnp.full((16,), np_lo, jnp.int32)
    cntv[pl.ds(16, 16)] = jnp.full((16,), np_hi, jnp.int32)

    @pl.when(c == 0)
    def _():
        pltpu.sync_copy(lsv, lists_hbm.at[s])
        pltpu.sync_copy(cntv, counts_hbm.at[s])


_partition = pl.kernel(
    _partition_body,
    out_type=[jax.ShapeDtypeStruct((NS, 4, CAP), jnp.int32),
              jax.ShapeDtypeStruct((NS, 32), jnp.int32)],
    mesh=plsc.VectorSubcoreMesh(core_axis_name="c", subcore_axis_name="s"),
    scratch_types=[
        pltpu.VMEM((EPS,), jnp.int32),
        pltpu.VMEM((EPS,), jnp.int32),
        pltpu.VMEM((4, CAP), jnp.int32),
        pltpu.VMEM((32,), jnp.int32),
    ],
)


def _seg_sum_body(h_hbm, lists_hbm, counts_hbm, agg_hbm,
                  lsv, ldv, idx0, idx1, idx2, rows0, rows1, rows2, zbuf,
                  cntv, shared, sem0, sem1, sem2, zsem):
    c = lax.axis_index("c")
    s = lax.axis_index("s")

    zeros16 = jnp.zeros((16,), jnp.float32)

    def zrow(r, carry):
        for j in range(HD // 16):
            zbuf[r, pl.ds(j * 16, 16)] = zeros16
        return carry

    lax.fori_loop(0, ZR, zrow, 0)
    pltpu.sync_copy(counts_hbm.at[s], cntv)

    for p in range(2):
        zb = s * APS
        descs = []
        for j in range(APS // ZR):
            descs.append(pltpu.async_copy(
                zbuf, shared.at[pl.ds(zb + j * ZR, ZR)], zsem))
        descs.append(pltpu.async_copy(
            zbuf.at[pl.ds(0, APS % ZR)],
            shared.at[pl.ds(zb + (APS // ZR) * ZR, APS % ZR)], zsem))
        pltpu.sync_copy(lists_hbm.at[s].at[2 * p], lsv)
        pltpu.sync_copy(lists_hbm.at[s].at[2 * p + 1], ldv)
        for d in descs:
            d.wait()
        plsc.subcore_barrier()

        ntrip = _lane0(cntv[pl.ds(p * 16, 16)])
        nch = ntrip * 3

        def build_idx(idxv, kk):
            base = kk * G
            for j in range(G // 16):
                idxv[pl.ds(j * 16, 16)] = ldv[pl.ds(base + j * 16, 16)]

        def start_gather(kk, rowsv, sem):
            pltpu.async_copy(h_hbm.at[c].at[lsv.at[pl.ds(kk * G, G)]],
                             rowsv, sem)

        @pl.when(ntrip > 0)
        def _():
            start_gather(0, rows0, sem0)
            start_gather(1, rows1, sem1)
            start_gather(2, rows2, sem2)

        def trip(i, carry):
            for b, (idxv, rowsv, sem) in enumerate(
                    ((idx0, rows0, sem0), (idx1, rows1, sem1),
                     (idx2, rows2, sem2))):
                kk = 3 * i + b
                pltpu.make_async_copy(h_hbm.at[c].at[lsv.at[pl.ds(0, G)]],
                                      rowsv, sem).wait()
                build_idx(idxv, kk)
                pltpu.sync_copy(rowsv, shared.at[idxv], add=True)

                @pl.when(kk + 3 < nch)
                def _():
                    start_gather(kk + 3, rowsv, sem)
            return carry

        lax.fori_loop(0, ntrip, trip, 0)
        plsc.subcore_barrier()

        pltpu.sync_copy(shared.at[pl.ds(s * WPS, WPS)],
                        agg_hbm.at[c].at[pl.ds(p * HALF + s * WPS, WPS)])
        plsc.subcore_barrier()


_seg_sum = pl.kernel(
    _seg_sum_body,
    out_type=jax.ShapeDtypeStruct((2, PN, HD), jnp.float32),
    mesh=plsc.VectorSubcoreMesh(core_axis_name="c", subcore_axis_name="s"),
    scratch_types=[
        pltpu.VMEM((CAP,), jnp.int32),
        pltpu.VMEM((CAP,), jnp.int32),
        pltpu.VMEM((G,), jnp.int32),
        pltpu.VMEM((G,), jnp.int32),
        pltpu.VMEM((G,), jnp.int32),
        pltpu.VMEM((G, HD), jnp.float32),
        pltpu.VMEM((G, HD), jnp.float32),
        pltpu.VMEM((G, HD), jnp.float32),
        pltpu.VMEM((ZR, HD), jnp.float32),
        pltpu.VMEM((32,), jnp.int32),
        pltpu.VMEM_SHARED((ACC, HD), jnp.float32),
        pltpu.SemaphoreType.DMA,
        pltpu.SemaphoreType.DMA,
        pltpu.SemaphoreType.DMA,
        pltpu.SemaphoreType.DMA,
    ],
)


BLK = 2000


def _gin_mlp_body(h, a, W1, b1, W2, b2, o, *, act_tanh):
    z = jnp.concatenate([h[0] + a[0], h[1] + a[1]], axis=1)
    t = jnp.dot(z, W1[...], preferred_element_type=jnp.float32) + b1[...]
    t = jnp.maximum(t, 0.0)
    t = jnp.dot(t, W2[...], preferred_element_type=jnp.float32) + b2[...]
    if act_tanh:
        t = jnp.tanh(t)
    o[0] = t[:, :HD]
    o[1] = t[:, HD:]


def _make_gin_mlp(act_tanh):
    pair = pl.BlockSpec((2, BLK, HD), lambda i: (0, i, 0))
    wspec = pl.BlockSpec((D, D), lambda i: (0, 0))
    bspec = pl.BlockSpec((1, D), lambda i: (0, 0))
    return pl.pallas_call(
        functools.partial(_gin_mlp_body, act_tanh=act_tanh),
        grid=(N // BLK,),
        in_specs=[pair, pair, wspec, bspec, wspec, bspec],
        out_specs=pair,
        out_shape=jax.ShapeDtypeStruct((2, N, HD), jnp.float32),
    )


_gin_mlp_tanh = _make_gin_mlp(True)


def _decoder_body(h, a, gW1, gb1, gW2, gb2,
                  dW1, db1, dW2, db2, dW3, db3, dW4, db4, dW5, db5,
                  gum, out):
    z = jnp.concatenate([h[0] + a[0], h[1] + a[1]], axis=1)
    t = jnp.maximum(jnp.dot(z, gW1[...], preferred_element_type=jnp.float32) + gb1[...], 0.0)
    t = jnp.dot(t, gW2[...], preferred_element_type=jnp.float32) + gb2[...]
    t = jnp.maximum(jnp.dot(t, dW1[...], preferred_element_type=jnp.float32) + db1[...], 0.0)
    t = jnp.maximum(jnp.dot(t, dW2[...], preferred_element_type=jnp.float32) + db2[...], 0.0)
    t = jnp.maximum(jnp.dot(t, dW3[...], preferred_element_type=jnp.float32) + db3[...], 0.0)
    t = jnp.dot(t, dW4[...], preferred_element_type=jnp.float32) + db4[...]
    t = jnp.dot(t, dW5[...], preferred_element_type=jnp.float32) + db5[...]
    t = t + gum[...]
    m = jnp.max(t, axis=1, keepdims=True)
    e = jnp.exp(t - m)
    out[...] = e / jnp.sum(e, axis=1, keepdims=True)


def _make_decoder():
    pair = pl.BlockSpec((2, BLK, HD), lambda i: (0, i, 0))
    w = pl.BlockSpec((D, D), lambda i: (0, 0))
    b = pl.BlockSpec((1, D), lambda i: (0, 0))
    wk = pl.BlockSpec((D, K), lambda i: (0, 0))
    bk = pl.BlockSpec((1, K), lambda i: (0, 0))
    wkk = pl.BlockSpec((K, K), lambda i: (0, 0))
    gspec = pl.BlockSpec((BLK, K), lambda i: (i, 0))
    return pl.pallas_call(
        _decoder_body,
        grid=(N // BLK,),
        in_specs=[pair, pair, w, b, w, b,
                  w, b, w, b, w, b, wk, bk, wkk, bk, gspec],
        out_specs=gspec,
        out_shape=jax.ShapeDtypeStruct((N, K), jnp.float32),
    )


_decoder = _make_decoder()


def kernel(x, edge_index, g1W1, g1b1, g1W2, g1b2, g2W1, g2b1, g2W2, g2b2,
           dW1, db1, dW2, db2, dW3, db3, dW4, db4, dW5, db5):
    src = edge_index[0].reshape(NS, EPS)
    dst = edge_index[1].reshape(NS, EPS)
    h = jnp.stack([x[:, :HD], x[:, HD:]], axis=0)

    lists, counts = _partition(src, dst)

    b1 = g1b1.reshape(1, D)
    b2 = g1b2.reshape(1, D)
    c1 = g2b1.reshape(1, D)
    c2 = g2b2.reshape(1, D)

    a = _seg_sum(h, lists, counts)
    h = _gin_mlp_tanh(h, a[:, :N], g1W1, b1, g1W2, b2)
    for _ in range(3):
        a = _seg_sum(h, lists, counts)
        h = _gin_mlp_tanh(h, a[:, :N], g2W1, c1, g2W2, c2)

    a = _seg_sum(h, lists, counts)
    u = jax.random.uniform(jax.random.key(42), (N, K), minval=1e-6, maxval=1.0 - 1e-6)
    gum = -jnp.log(-jnp.log(u))
    return _decoder(h, a[:, :N], g2W1, c1, g2W2, c2,
                    dW1, db1.reshape(1, D), dW2, db2.reshape(1, D),
                    dW3, db3.reshape(1, D), dW4, db4.reshape(1, K),
                    dW5, db5.reshape(1, K), gum)
